# baseline (speedup 1.0000x reference)
import functools

import jax
import jax.numpy as jnp
from jax import lax
from jax.experimental import pallas as pl
from jax.experimental.pallas import tpu as pltpu
from jax.experimental.pallas import tpu_sc as plsc

B = 1024
S = 200
DIM = 64

_info = plsc.get_sparse_core_info()
NC, NS, L = _info.num_cores, _info.num_subcores, _info.num_lanes
NW = NC * NS
ROWS_PER_W = B // NW
CHUNK = 40
HPR = S // CHUNK
NCHUNKS = ROWS_PER_W * HPR
VPR = DIM // 16
RING = 4
LEAD = 2


def _make_kernel():
  mesh = plsc.VectorSubcoreMesh(core_axis_name="c", subcore_axis_name="s")

  rows_scratch = [pltpu.VMEM((CHUNK, DIM), jnp.float32) for _ in range(RING)]
  sem_scratch = [pltpu.SemaphoreType.DMA for _ in range(2 * RING)]

  @functools.partial(
      pl.kernel,
      mesh=mesh,
      compiler_params=pltpu.CompilerParams(use_tc_tiling_on_sc=False),
      out_type=jax.ShapeDtypeStruct((B, S, DIM), jnp.float32),
      scratch_types=[
          pltpu.VMEM((S, DIM), jnp.float32),
          pltpu.VMEM((ROWS_PER_W, S), jnp.int32),
      ] + rows_scratch + sem_scratch,
  )
  def k(ids_hbm, table_hbm, pos_hbm, out_hbm, pos_v, idx_v, *rest):
    bufs = rest[:RING]
    gsems = rest[RING:2 * RING]
    ssems = rest[2 * RING:]
    wid = lax.axis_index("s") * NC + lax.axis_index("c")
    pltpu.sync_copy(pos_hbm.at[0, pl.ds(0, S)], pos_v)
    row0 = wid * ROWS_PER_W
    pltpu.sync_copy(ids_hbm.at[pl.ds(row0, ROWS_PER_W)], idx_v)

    def issue_gather(c, b):
      r = lax.div(c, HPR)
      h = lax.rem(c, HPR)
      pltpu.async_copy(
          table_hbm.at[idx_v.at[r, pl.ds(h * CHUNK, CHUNK)]], bufs[b],
          gsems[b])

    def wait_gather(b):
      pltpu.make_async_copy(
          table_hbm.at[pl.ds(0, CHUNK)], bufs[b], gsems[b]).wait()

    def wait_store(b):
      pltpu.make_async_copy(
          bufs[b], out_hbm.at[0, pl.ds(0, CHUNK), :], ssems[b]).wait()

    for c in range(LEAD):
      issue_gather(c, c % RING)

    def step(c, b):
      @pl.when(c + LEAD < NCHUNKS)
      def _():
        bp = (b + LEAD) % RING

        @pl.when(c >= RING - LEAD)
        def _():
          wait_store(bp)

        issue_gather(c + LEAD, bp)

      wait_gather(b)
      cur = bufs[b]
      r = lax.div(c, HPR)
      h = lax.rem(c, HPR)
      prow0 = h * CHUNK

      def row_body(rr, carry2):
        pr = prow0 + rr
        for kk in range(VPR):
          sl = pl.ds(kk * 16, 16)
          plsc.addupdate(cur.at[rr, sl], pos_v[pr, sl])
        return carry2

      lax.fori_loop(0, CHUNK, row_body, 0, unroll=8)
      pltpu.async_copy(
          cur, out_hbm.at[row0 + r, pl.ds(prow0, CHUNK), :], ssems[b])

    def ring_body(j, carry):
      for b in range(RING):
        step(j * RING + b, b)
      return carry

    lax.fori_loop(0, NCHUNKS // RING, ring_body, 0)

    for b in range(RING):
      wait_store(b)

  return k


_kernel = _make_kernel()


def kernel(input_ids, token_table, position_embedding):
  return _kernel(input_ids.astype(jnp.int32), token_table,
                 position_embedding)


# speedup vs baseline: 1.0109x; 1.0109x over previous
import functools

import jax
import jax.numpy as jnp
from jax import lax
from jax.experimental import pallas as pl
from jax.experimental.pallas import tpu as pltpu
from jax.experimental.pallas import tpu_sc as plsc

B = 1024
S = 200
DIM = 64

_info = plsc.get_sparse_core_info()
NC, NS, L = _info.num_cores, _info.num_subcores, _info.num_lanes
NW = NC * NS
ROWS_PER_W = B // NW
TOK_PER_W = ROWS_PER_W * S
CHUNK = 80
NCHUNKS = TOK_PER_W // CHUNK
VPR = DIM // 16
RING = 4
LEAD = 2
NIDV = TOK_PER_W // 16


def _make_kernel():
  mesh = plsc.VectorSubcoreMesh(core_axis_name="c", subcore_axis_name="s")

  rows_scratch = [pltpu.VMEM((CHUNK, DIM), jnp.float32) for _ in range(RING)]
  sem_scratch = [pltpu.SemaphoreType.DMA for _ in range(2 * RING)]

  @functools.partial(
      pl.kernel,
      mesh=mesh,
      compiler_params=pltpu.CompilerParams(
          use_tc_tiling_on_sc=False, needs_layout_passes=False),
      out_type=jax.ShapeDtypeStruct((B * S, DIM), jnp.float32),
      scratch_types=[
          pltpu.VMEM((S, DIM), jnp.float32),
          pltpu.VMEM((ROWS_PER_W, S), jnp.float32),
          pltpu.VMEM((TOK_PER_W,), jnp.int32),
      ] + rows_scratch + sem_scratch,
  )
  def k(idsf_hbm, table_hbm, pos_hbm, out_hbm, pos_v, idsf_v, idx_v, *rest):
    bufs = rest[:RING]
    gsems = rest[RING:2 * RING]
    ssems = rest[2 * RING:]
    wid = lax.axis_index("s") * NC + lax.axis_index("c")
    pltpu.sync_copy(pos_hbm.at[0, pl.ds(0, S)], pos_v)
    row0 = wid * ROWS_PER_W
    pltpu.sync_copy(idsf_hbm.at[pl.ds(row0, ROWS_PER_W)], idsf_v)
    base0 = wid * TOK_PER_W

    def cvt_body(v, carry):
      r = lax.div(v, S // 16)
      cc = lax.rem(v, S // 16) * 16
      x = idsf_v[r, pl.ds(cc, 16)]
      idx_v[pl.ds(v * 16, 16)] = plsc.bitcast(x, jnp.int32)
      return carry

    lax.fori_loop(0, NIDV, cvt_body, 0, unroll=8)

    def issue_gather(c, b):
      pltpu.async_copy(
          table_hbm.at[idx_v.at[pl.ds(c * CHUNK, CHUNK)]], bufs[b], gsems[b])

    def wait_gather(b):
      pltpu.make_async_copy(
          table_hbm.at[pl.ds(0, CHUNK)], bufs[b], gsems[b]).wait()

    def wait_store(b):
      pltpu.make_async_copy(
          bufs[b], out_hbm.at[pl.ds(0, CHUNK)], ssems[b]).wait()

    for c in range(LEAD):
      issue_gather(c, c % RING)

    def step(c, b):
      @pl.when(c + LEAD < NCHUNKS)
      def _():
        bp = (b + LEAD) % RING

        @pl.when(c >= RING - LEAD)
        def _():
          wait_store(bp)

        issue_gather(c + LEAD, bp)

      wait_gather(b)
      cur = bufs[b]
      prow0 = lax.rem(c * CHUNK, S)

      def row_body(rr, carry2):
        pr = lax.rem(prow0 + rr, S)
        for kk in range(VPR):
          sl = pl.ds(kk * 16, 16)
          plsc.addupdate(cur.at[rr, sl], pos_v[pr, sl])
        return carry2

      lax.fori_loop(0, CHUNK, row_body, 0, unroll=8)
      pltpu.async_copy(
          cur, out_hbm.at[pl.ds(base0 + c * CHUNK, CHUNK)], ssems[b])

    def ring_body(j, carry):
      for b in range(RING):
        step(j * RING + b, b)
      return carry

    lax.fori_loop(0, NCHUNKS // RING, ring_body, 0)

    for b in range(RING):
      wait_store(b)

  return k


_kernel = _make_kernel()


def kernel(input_ids, token_table, position_embedding):
  Bq, Sq = input_ids.shape
  ids_f = lax.bitcast_convert_type(input_ids.astype(jnp.int32), jnp.float32)
  out = _kernel(ids_f, token_table, position_embedding)
  return out.reshape(Bq, Sq, DIM)


# speedup vs baseline: 1.0117x; 1.0008x over previous
import functools

import jax
import jax.numpy as jnp
from jax import lax
from jax.experimental import pallas as pl
from jax.experimental.pallas import tpu as pltpu
from jax.experimental.pallas import tpu_sc as plsc

B = 1024
S = 200
DIM = 64

_info = plsc.get_sparse_core_info()
NC, NS, L = _info.num_cores, _info.num_subcores, _info.num_lanes
NW = NC * NS
ROWS_PER_W = B // NW
TOK_PER_W = ROWS_PER_W * S
CHUNK = 80
NCHUNKS = TOK_PER_W // CHUNK
VPR = DIM // 16
RING = 4
LEAD = 2
NIDV = TOK_PER_W // 16


def _make_kernel():
  mesh = plsc.VectorSubcoreMesh(core_axis_name="c", subcore_axis_name="s")

  rows_scratch = [pltpu.VMEM((CHUNK, DIM), jnp.float32) for _ in range(RING)]
  sem_scratch = [pltpu.SemaphoreType.DMA for _ in range(2 * RING)]

  @functools.partial(
      pl.kernel,
      mesh=mesh,
      compiler_params=pltpu.CompilerParams(
          use_tc_tiling_on_sc=False, needs_layout_passes=False),
      out_type=jax.ShapeDtypeStruct((B * S, DIM), jnp.float32),
      scratch_types=[
          pltpu.VMEM((S, DIM), jnp.float32),
          pltpu.VMEM((TOK_PER_W,), jnp.float32),
          pltpu.VMEM((TOK_PER_W,), jnp.int32),
      ] + rows_scratch + sem_scratch,
  )
  def k(idsf_hbm, table_hbm, pos_hbm, out_hbm, pos_v, idsf_v, idx_v, *rest):
    bufs = rest[:RING]
    gsems = rest[RING:2 * RING]
    ssems = rest[2 * RING:]
    wid = lax.axis_index("s") * NC + lax.axis_index("c")
    pltpu.sync_copy(pos_hbm.at[0, pl.ds(0, S)], pos_v)
    row0 = wid * ROWS_PER_W
    base0 = wid * TOK_PER_W
    for r in range(ROWS_PER_W):
      pltpu.async_copy(idsf_hbm.at[row0 + r],
                       idsf_v.at[pl.ds(r * S, S)], gsems[0])
    for r in range(ROWS_PER_W):
      pltpu.make_async_copy(idsf_hbm.at[0],
                            idsf_v.at[pl.ds(0, S)], gsems[0]).wait()

    def cvt_body(v, carry):
      x = idsf_v[pl.ds(v * 16, 16)]
      idx_v[pl.ds(v * 16, 16)] = plsc.bitcast(x, jnp.int32) & jnp.int32(
          0x007FFFFF)
      return carry

    lax.fori_loop(0, NIDV, cvt_body, 0, unroll=8)

    def issue_gather(c, b):
      pltpu.async_copy(
          table_hbm.at[idx_v.at[pl.ds(c * CHUNK, CHUNK)]], bufs[b], gsems[b])

    def wait_gather(b):
      pltpu.make_async_copy(
          table_hbm.at[pl.ds(0, CHUNK)], bufs[b], gsems[b]).wait()

    def wait_store(b):
      pltpu.make_async_copy(
          bufs[b], out_hbm.at[pl.ds(0, CHUNK)], ssems[b]).wait()

    for c in range(LEAD):
      issue_gather(c, c % RING)

    def step(c, b):
      @pl.when(c + LEAD < NCHUNKS)
      def _():
        bp = (b + LEAD) % RING

        @pl.when(c >= RING - LEAD)
        def _():
          wait_store(bp)

        issue_gather(c + LEAD, bp)

      wait_gather(b)
      cur = bufs[b]
      prow0 = lax.rem(c * CHUNK, S)

      def row_body(rr, carry2):
        pr = lax.rem(prow0 + rr, S)
        for kk in range(VPR):
          sl = pl.ds(kk * 16, 16)
          plsc.addupdate(cur.at[rr, sl], pos_v[pr, sl])
        return carry2

      lax.fori_loop(0, CHUNK, row_body, 0, unroll=8)
      pltpu.async_copy(
          cur, out_hbm.at[pl.ds(base0 + c * CHUNK, CHUNK)], ssems[b])

    def ring_body(j, carry):
      for b in range(RING):
        step(j * RING + b, b)
      return carry

    lax.fori_loop(0, NCHUNKS // RING, ring_body, 0)

    for b in range(RING):
      wait_store(b)

  return k


_kernel = _make_kernel()


def kernel(input_ids, token_table, position_embedding):
  Bq, Sq = input_ids.shape
  ids_tagged = input_ids.astype(jnp.int32) | jnp.int32(0x4B000000)
  ids_f = lax.bitcast_convert_type(ids_tagged, jnp.float32)
  out = _kernel(ids_f, token_table, position_embedding)
  return out.reshape(Bq, Sq, DIM)
